# Initial kernel scaffold; baseline (speedup 1.0000x reference)
import functools

import jax
import jax.numpy as jnp
from jax import lax
from jax.experimental import pallas as pl
from jax.experimental.pallas import tpu as pltpu
from jax.experimental.pallas import tpu_sc as plsc

N_NODES = 10000
N_EDGES = 320000
D = 128

NC = 2
NS = 16
NW = NC * NS

E_PER_W = N_EDGES // NW
CHUNK = 80
N_CHUNKS = E_PER_W // CHUNK
ROWS_PER_TILE = N_NODES // NS
ZROWS = 125
CNT_W = 16


def _sc_aggregate_body(x_hbm, src_hbm, dst_hbm, acc_out, cnt_out,
                       src_v, dst_v, rows_v, ones_v, zb, zb2, acc_sh, cnt_sh):
    c = lax.axis_index("c")
    s = lax.axis_index("s")
    wid = c * NS + s

    def zrow(r, _):
        for k in range(D // 16):
            zb[r, pl.ds(k * 16, 16)] = jnp.zeros((16,), jnp.float32)
        return 0
    lax.fori_loop(0, ZROWS, zrow, 0)

    def orow(r, _):
        ones_v[r, pl.ds(0, 16)] = jnp.ones((16,), jnp.float32)
        return 0
    lax.fori_loop(0, CHUNK, orow, 0)

    def z2row(r, _):
        zb2[r, pl.ds(0, 16)] = jnp.zeros((16,), jnp.float32)
        return 0
    lax.fori_loop(0, ROWS_PER_TILE, z2row, 0)

    row0 = s * ROWS_PER_TILE
    for k in range(ROWS_PER_TILE // ZROWS):
        pltpu.sync_copy(zb, acc_sh.at[pl.ds(row0 + k * ZROWS, ZROWS)])
    pltpu.sync_copy(zb2, cnt_sh.at[pl.ds(row0, ROWS_PER_TILE)])
    plsc.subcore_barrier()

    def body(j, _):
        base = pl.multiple_of(wid * E_PER_W + j * CHUNK, CHUNK)
        pltpu.sync_copy(src_hbm.at[pl.ds(base, CHUNK)], src_v)
        pltpu.sync_copy(dst_hbm.at[pl.ds(base, CHUNK)], dst_v)
        pltpu.sync_copy(x_hbm.at[src_v], rows_v)
        pltpu.sync_copy(rows_v, acc_sh.at[dst_v], add=True)
        pltpu.sync_copy(ones_v, cnt_sh.at[dst_v], add=True)
        return 0
    lax.fori_loop(0, N_CHUNKS, body, 0)
    plsc.subcore_barrier()

    for k in range(ROWS_PER_TILE // ZROWS):
        r = row0 + k * ZROWS
        pltpu.sync_copy(acc_sh.at[pl.ds(r, ZROWS)], zb)
        pltpu.sync_copy(zb, acc_out.at[c, pl.ds(r, ZROWS)])
    pltpu.sync_copy(cnt_sh.at[pl.ds(row0, ROWS_PER_TILE)], zb2)
    pltpu.sync_copy(zb2, cnt_out.at[c, pl.ds(row0, ROWS_PER_TILE)])


_sc_aggregate = functools.partial(
    pl.kernel,
    mesh=plsc.VectorSubcoreMesh(core_axis_name="c", subcore_axis_name="s"),
    out_type=[
        jax.ShapeDtypeStruct((NC, N_NODES, D), jnp.float32),
        jax.ShapeDtypeStruct((NC, N_NODES, CNT_W), jnp.float32),
    ],
    scratch_types=[
        pltpu.VMEM((CHUNK,), jnp.int32),
        pltpu.VMEM((CHUNK,), jnp.int32),
        pltpu.VMEM((CHUNK, D), jnp.float32),
        pltpu.VMEM((CHUNK, CNT_W), jnp.float32),
        pltpu.VMEM((ZROWS, D), jnp.float32),
        pltpu.VMEM((ROWS_PER_TILE, CNT_W), jnp.float32),
        pltpu.VMEM_SHARED((N_NODES, D), jnp.float32),
        pltpu.VMEM_SHARED((N_NODES, CNT_W), jnp.float32),
    ],
)(_sc_aggregate_body)


ROWS_PER_BLK = 1000


def _tc_finish_body(acc_ref, cnt_ref, w_ref, b_ref, out_ref):
    s = acc_ref[0] + acc_ref[1]
    cnt = cnt_ref[0] + cnt_ref[1]
    deg = cnt[:, 0:1]
    mean = s / jnp.maximum(deg, 1.0)
    h = lax.dot_general(mean, w_ref[...], (((1,), (1,)), ((), ())),
                        preferred_element_type=jnp.float32)
    out_ref[...] = h + b_ref[...]


def _tc_finish(acc, cnt, w, b):
    return pl.pallas_call(
        _tc_finish_body,
        grid=(N_NODES // ROWS_PER_BLK,),
        in_specs=[
            pl.BlockSpec((NC, ROWS_PER_BLK, D), lambda i: (0, i, 0)),
            pl.BlockSpec((NC, ROWS_PER_BLK, CNT_W), lambda i: (0, i, 0)),
            pl.BlockSpec((D, D), lambda i: (0, 0)),
            pl.BlockSpec((1, D), lambda i: (0, 0)),
        ],
        out_specs=pl.BlockSpec((ROWS_PER_BLK, D), lambda i: (i, 0)),
        out_shape=jax.ShapeDtypeStruct((N_NODES, D), jnp.float32),
    )(acc, cnt, w, b)


@jax.jit
def kernel(x, edge_index, W, b):
    ei = edge_index.astype(jnp.int32)
    src = ei[0]
    dst = ei[1]
    acc, cnt = _sc_aggregate(x, src, dst)
    return _tc_finish(acc, cnt, W, b.reshape(1, D))


# speedup vs baseline: 3.6200x; 3.6200x over previous
import functools

import jax
import jax.numpy as jnp
from jax import lax
from jax.experimental import pallas as pl
from jax.experimental.pallas import tpu as pltpu
from jax.experimental.pallas import tpu_sc as plsc

N_NODES = 10000
N_EDGES = 320000
D = 128
DH = D // 2

NC = 2
NS = 16

E_PER_T = N_EDGES // NS
E_PER_CT = N_EDGES // (NC * NS)
CHUNK = 80
N_PAD = 10240
ROWS_PER_TILE = N_PAD // NS
ZROWS = 128
CNT_W = 16


def _sc_aggregate_body(xcat_hbm, srcs_hbm, dst_hbm, acc_out, cnt_out,
                       src_v, dst_v, rows_v, ones_v, zb, zb2, acc_sh, cnt_sh):
    c = lax.axis_index("c")
    s = lax.axis_index("s")

    def zrow(r, _):
        for k in range(DH // 16):
            zb[r, pl.ds(k * 16, 16)] = jnp.zeros((16,), jnp.float32)
        return 0
    lax.fori_loop(0, ZROWS, zrow, 0)

    def orow(r, _):
        ones_v[r, pl.ds(0, 16)] = jnp.ones((16,), jnp.float32)
        return 0
    lax.fori_loop(0, CHUNK, orow, 0)

    def z2row(r, _):
        zb2[r, pl.ds(0, 16)] = jnp.zeros((16,), jnp.float32)
        return 0
    lax.fori_loop(0, ROWS_PER_TILE, z2row, 0)

    row0 = s * ROWS_PER_TILE
    for k in range(ROWS_PER_TILE // ZROWS):
        pltpu.sync_copy(zb, acc_sh.at[pl.ds(row0 + k * ZROWS, ZROWS)])
    pltpu.sync_copy(zb2, cnt_sh.at[pl.ds(row0, ROWS_PER_TILE)])
    plsc.subcore_barrier()

    def body(j, _):
        base = pl.multiple_of(c * N_EDGES + s * E_PER_T + j * CHUNK, CHUNK)
        dbase = pl.multiple_of(s * E_PER_T + j * CHUNK, CHUNK)
        pltpu.sync_copy(srcs_hbm.at[pl.ds(base, CHUNK)], src_v)
        pltpu.sync_copy(dst_hbm.at[pl.ds(dbase, CHUNK)], dst_v)
        pltpu.sync_copy(xcat_hbm.at[src_v], rows_v)
        pltpu.sync_copy(rows_v, acc_sh.at[dst_v], add=True)
        return 0
    lax.fori_loop(0, E_PER_T // CHUNK, body, 0)

    def cbody(j, _):
        base = pl.multiple_of((c * NS + s) * E_PER_CT + j * CHUNK, CHUNK)
        pltpu.sync_copy(dst_hbm.at[pl.ds(base, CHUNK)], dst_v)
        pltpu.sync_copy(ones_v, cnt_sh.at[dst_v], add=True)
        return 0
    lax.fori_loop(0, E_PER_CT // CHUNK, cbody, 0)
    plsc.subcore_barrier()

    for k in range(ROWS_PER_TILE // ZROWS):
        r = row0 + k * ZROWS
        pltpu.sync_copy(acc_sh.at[pl.ds(r, ZROWS)], zb)
        pltpu.sync_copy(zb, acc_out.at[c, pl.ds(r, ZROWS)])
    pltpu.sync_copy(cnt_sh.at[pl.ds(row0, ROWS_PER_TILE)], zb2)
    pltpu.sync_copy(zb2, cnt_out.at[c, pl.ds(row0, ROWS_PER_TILE)])


_sc_aggregate = functools.partial(
    pl.kernel,
    mesh=plsc.VectorSubcoreMesh(core_axis_name="c", subcore_axis_name="s"),
    compiler_params=pltpu.CompilerParams(use_tc_tiling_on_sc=False),
    out_type=[
        jax.ShapeDtypeStruct((NC, N_PAD, DH), jnp.float32),
        jax.ShapeDtypeStruct((NC, N_PAD, CNT_W), jnp.float32),
    ],
    scratch_types=[
        pltpu.VMEM((CHUNK,), jnp.int32),
        pltpu.VMEM((CHUNK,), jnp.int32),
        pltpu.VMEM((CHUNK, DH), jnp.float32),
        pltpu.VMEM((CHUNK, CNT_W), jnp.float32),
        pltpu.VMEM((ZROWS, DH), jnp.float32),
        pltpu.VMEM((ROWS_PER_TILE, CNT_W), jnp.float32),
        pltpu.VMEM_SHARED((N_PAD, DH), jnp.float32),
        pltpu.VMEM_SHARED((N_PAD, CNT_W), jnp.float32),
    ],
)(_sc_aggregate_body)


ROWS_PER_BLK = 1000


def _tc_finish_body(acc_ref, cnt_ref, w_ref, b_ref, out_ref):
    s = jnp.concatenate([acc_ref[0], acc_ref[1]], axis=1)
    cnt = cnt_ref[0] + cnt_ref[1]
    deg = cnt[:, 0:1]
    mean = s / jnp.maximum(deg, 1.0)
    h = lax.dot_general(mean, w_ref[...], (((1,), (1,)), ((), ())),
                        preferred_element_type=jnp.float32)
    out_ref[...] = h + b_ref[...]


def _tc_finish(acc, cnt, w, b):
    return pl.pallas_call(
        _tc_finish_body,
        grid=(N_NODES // ROWS_PER_BLK,),
        in_specs=[
            pl.BlockSpec((NC, ROWS_PER_BLK, DH), lambda i: (0, i, 0)),
            pl.BlockSpec((NC, ROWS_PER_BLK, CNT_W), lambda i: (0, i, 0)),
            pl.BlockSpec((D, D), lambda i: (0, 0)),
            pl.BlockSpec((1, D), lambda i: (0, 0)),
        ],
        out_specs=pl.BlockSpec((ROWS_PER_BLK, D), lambda i: (i, 0)),
        out_shape=jax.ShapeDtypeStruct((N_NODES, D), jnp.float32),
    )(acc, cnt, w, b)


@jax.jit
def kernel(x, edge_index, W, b):
    ei = edge_index.astype(jnp.int32)
    src = ei[0]
    dst = ei[1]
    xcat = jnp.concatenate([x[:, :DH], x[:, DH:]], axis=0)
    srcs = jnp.concatenate([src, src + N_NODES])
    acc, cnt = _sc_aggregate(xcat, srcs, dst)
    return _tc_finish(acc, cnt, W, b.reshape(1, D))


# speedup vs baseline: 5.5203x; 1.5250x over previous
import functools

import jax
import jax.numpy as jnp
from jax import lax
from jax.experimental import pallas as pl
from jax.experimental.pallas import tpu as pltpu
from jax.experimental.pallas import tpu_sc as plsc

N_NODES = 10000
N_EDGES = 320000
D = 128
DH = D // 2

NC = 2
NS = 16

E_PER_T = N_EDGES // NS
CHUNK = 128
E_PAD_T = 20480
NCH = E_PAD_T // CHUNK
DUMMY = N_NODES
N_PAD = 10240
ROWS_PER_TILE = N_PAD // NS
ZROWS = 128
CNT_W = 16


def _sc_aggregate_body(xcat_hbm, pidx_hbm, acc_out, cnt_out,
                       idx_a, idx_b, rows_a, rows_b, ones_v, zb, zb2, sem,
                       acc_sh, cnt_sh):
    c = lax.axis_index("c")
    s = lax.axis_index("s")
    r0 = (c * NS + s) * NCH

    def zrow(r, _):
        for k in range(DH // 16):
            zb[r, pl.ds(k * 16, 16)] = jnp.zeros((16,), jnp.float32)
        return 0
    lax.fori_loop(0, ZROWS, zrow, 0)

    def orow(r, _):
        ones_v[r, pl.ds(0, 16)] = jnp.ones((16,), jnp.float32)
        return 0
    lax.fori_loop(0, CHUNK, orow, 0)

    def z2row(r, _):
        zb2[r, pl.ds(0, 16)] = jnp.zeros((16,), jnp.float32)
        return 0
    lax.fori_loop(0, ROWS_PER_TILE, z2row, 0)

    row0 = s * ROWS_PER_TILE
    for k in range(ROWS_PER_TILE // ZROWS):
        pltpu.sync_copy(zb, acc_sh.at[pl.ds(row0 + k * ZROWS, ZROWS)])
    pltpu.sync_copy(zb2, cnt_sh.at[pl.ds(row0, ROWS_PER_TILE)])
    plsc.subcore_barrier()

    def half(j, idx_x, idx_y, rows_x, rows_y, gx, gy, ix, iy):
        nxt = j + 1

        @pl.when(nxt < NCH)
        def _():
            pltpu.make_async_copy(pidx_hbm.at[r0 + nxt], idx_y, sem.at[iy]).wait()
            pltpu.async_copy(xcat_hbm.at[idx_y.at[0]], rows_y, sem.at[gy])

        pltpu.make_async_copy(xcat_hbm.at[idx_x.at[0]], rows_x, sem.at[gx]).wait()
        pltpu.sync_copy(rows_x, acc_sh.at[idx_x.at[1]], add=True)

        @pl.when((j < NCH // 2) == (c == 0))
        def _():
            pltpu.sync_copy(ones_v, cnt_sh.at[idx_x.at[1]], add=True)

        @pl.when(j + 2 < NCH)
        def _():
            pltpu.async_copy(pidx_hbm.at[r0 + j + 2], idx_x, sem.at[ix])

    pltpu.sync_copy(pidx_hbm.at[r0], idx_a)
    pltpu.async_copy(xcat_hbm.at[idx_a.at[0]], rows_a, sem.at[0])
    pltpu.async_copy(pidx_hbm.at[r0 + 1], idx_b, sem.at[3])

    def body(j2, _):
        j = j2 * 2
        half(j, idx_a, idx_b, rows_a, rows_b, 0, 1, 2, 3)
        half(j + 1, idx_b, idx_a, rows_b, rows_a, 1, 0, 3, 2)
        return 0
    lax.fori_loop(0, NCH // 2, body, 0)
    plsc.subcore_barrier()

    for k in range(ROWS_PER_TILE // ZROWS):
        r = row0 + k * ZROWS
        pltpu.sync_copy(acc_sh.at[pl.ds(r, ZROWS)], zb)
        pltpu.sync_copy(zb, acc_out.at[c, pl.ds(r, ZROWS)])
    pltpu.sync_copy(cnt_sh.at[pl.ds(row0, ROWS_PER_TILE)], zb2)
    pltpu.sync_copy(zb2, cnt_out.at[c, pl.ds(row0, ROWS_PER_TILE)])


_sc_aggregate = functools.partial(
    pl.kernel,
    mesh=plsc.VectorSubcoreMesh(core_axis_name="c", subcore_axis_name="s"),
    compiler_params=pltpu.CompilerParams(use_tc_tiling_on_sc=False),
    out_type=[
        jax.ShapeDtypeStruct((NC, N_PAD, DH), jnp.float32),
        jax.ShapeDtypeStruct((NC, N_PAD, CNT_W), jnp.float32),
    ],
    scratch_types=[
        pltpu.VMEM((2, CHUNK), jnp.int32),
        pltpu.VMEM((2, CHUNK), jnp.int32),
        pltpu.VMEM((CHUNK, DH), jnp.float32),
        pltpu.VMEM((CHUNK, DH), jnp.float32),
        pltpu.VMEM((CHUNK, CNT_W), jnp.float32),
        pltpu.VMEM((ZROWS, DH), jnp.float32),
        pltpu.VMEM((ROWS_PER_TILE, CNT_W), jnp.float32),
        pltpu.SemaphoreType.DMA((4,)),
        pltpu.VMEM_SHARED((N_PAD, DH), jnp.float32),
        pltpu.VMEM_SHARED((N_PAD, CNT_W), jnp.float32),
    ],
)(_sc_aggregate_body)


ROWS_PER_BLK = 1000


def _tc_finish_body(acc_ref, cnt_ref, w_ref, b_ref, out_ref):
    s = jnp.concatenate([acc_ref[0], acc_ref[1]], axis=1)
    cnt = cnt_ref[0] + cnt_ref[1]
    deg = cnt[:, 0:1]
    mean = s / jnp.maximum(deg, 1.0)
    h = lax.dot_general(mean, w_ref[...], (((1,), (1,)), ((), ())),
                        preferred_element_type=jnp.float32)
    out_ref[...] = h + b_ref[...]


def _tc_finish(acc, cnt, w, b):
    return pl.pallas_call(
        _tc_finish_body,
        grid=(N_NODES // ROWS_PER_BLK,),
        in_specs=[
            pl.BlockSpec((NC, ROWS_PER_BLK, DH), lambda i: (0, i, 0)),
            pl.BlockSpec((NC, ROWS_PER_BLK, CNT_W), lambda i: (0, i, 0)),
            pl.BlockSpec((D, D), lambda i: (0, 0)),
            pl.BlockSpec((1, D), lambda i: (0, 0)),
        ],
        out_specs=pl.BlockSpec((ROWS_PER_BLK, D), lambda i: (i, 0)),
        out_shape=jax.ShapeDtypeStruct((N_NODES, D), jnp.float32),
    )(acc, cnt, w, b)


@jax.jit
def kernel(x, edge_index, W, b):
    ei = edge_index.astype(jnp.int32)
    src = ei[0]
    dst = ei[1]
    xcat = jnp.concatenate([x[:, :DH], x[:, DH:]], axis=0)
    src_t = jnp.pad(src.reshape(NS, E_PER_T), ((0, 0), (0, E_PAD_T - E_PER_T)))
    dst_t = jnp.pad(dst.reshape(NS, E_PER_T), ((0, 0), (0, E_PAD_T - E_PER_T)),
                    constant_values=DUMMY)
    src_t = src_t.reshape(NS, NCH, CHUNK)
    dst_t = dst_t.reshape(NS, NCH, CHUNK)
    gidx = src_t[None] + (jnp.arange(NC, dtype=jnp.int32) * N_NODES)[:, None, None, None]
    dstb = jnp.broadcast_to(dst_t[None], (NC, NS, NCH, CHUNK))
    pidx = jnp.stack([gidx, dstb], axis=3).reshape(NC * NS * NCH, 2, CHUNK)
    acc, cnt = _sc_aggregate(xcat, pidx)
    return _tc_finish(acc, cnt, W, b.reshape(1, D))


# speedup vs baseline: 10.0916x; 1.8281x over previous
import functools

import jax
import jax.numpy as jnp
from jax import lax
from jax.experimental import pallas as pl
from jax.experimental.pallas import tpu as pltpu
from jax.experimental.pallas import tpu_sc as plsc

N_NODES = 10000
N_EDGES = 320000
D = 128
DH = D // 2

NC = 2
NS = 16

E_PER_T = N_EDGES // NS
CHUNK = 128
NCHF = E_PER_T // CHUNK
REM = E_PER_T - NCHF * CHUNK
CHALF = NCHF // 2
N_PAD = 10240
ROWS_PER_TILE = N_PAD // NS
ZROWS = 128
CNT_W = 16


def _sc_aggregate_body(xv_hbm, src_hbm, dst_hbm, acc_out, cnt_out,
                       sbufs, gbufs, dbufs, rowss, srem, grem, drem, rrem,
                       ones_v, zb, zb2, sem, acc_sh, cnt_sh):
    c = lax.axis_index("c")
    s = lax.axis_index("s")
    ebase = s * E_PER_T

    def zrow(r, _):
        for k in range(DH // 16):
            zb[r, pl.ds(k * 16, 16)] = jnp.zeros((16,), jnp.float32)
        return 0
    lax.fori_loop(0, ZROWS, zrow, 0)

    def orow(r, _):
        ones_v[r, pl.ds(0, 16)] = jnp.ones((16,), jnp.float32)
        return 0
    lax.fori_loop(0, CHUNK, orow, 0)

    def z2row(r, _):
        zb2[r, pl.ds(0, 16)] = jnp.zeros((16,), jnp.float32)
        return 0
    lax.fori_loop(0, ROWS_PER_TILE, z2row, 0)

    row0 = s * ROWS_PER_TILE
    for k in range(ROWS_PER_TILE // ZROWS):
        pltpu.sync_copy(zb, acc_sh.at[pl.ds(row0 + k * ZROWS, ZROWS)])
    pltpu.sync_copy(zb2, cnt_sh.at[pl.ds(row0, ROWS_PER_TILE)])
    plsc.subcore_barrier()

    def cpred(j):
        return (j < CHALF) == (c == 0)

    def load_idx(j, slot):
        base = pl.multiple_of(ebase + j * CHUNK, CHUNK)
        pltpu.async_copy(src_hbm.at[pl.ds(base, CHUNK)], sbufs[slot], sem.at[slot])
        pltpu.async_copy(dst_hbm.at[pl.ds(base, CHUNK)], dbufs[slot], sem.at[slot])

    def wait_idx(j, slot):
        base = pl.multiple_of(ebase + j * CHUNK, CHUNK)
        pltpu.make_async_copy(src_hbm.at[pl.ds(base, CHUNK)], sbufs[slot], sem.at[slot]).wait()
        pltpu.make_async_copy(dst_hbm.at[pl.ds(base, CHUNK)], dbufs[slot], sem.at[slot]).wait()

    def compute_gidx(slot):
        for k in range(CHUNK // 16):
            v = sbufs[slot][pl.ds(k * 16, 16)]
            gbufs[slot][pl.ds(k * 16, 16)] = v + v + c

    def start_gather(slot):
        pltpu.async_copy(xv_hbm.at[gbufs[slot]], rowss[slot], sem.at[4 + slot])

    def wait_gather(slot):
        pltpu.make_async_copy(xv_hbm.at[gbufs[slot]], rowss[slot], sem.at[4 + slot]).wait()

    def half(j, x_):
        w, z = (x_ + 1) % 4, (x_ + 2) % 4
        wait_gather(x_)
        pltpu.async_copy(rowss[x_], acc_sh.at[dbufs[x_]], sem.at[8 + x_], add=True)

        @pl.when(cpred(j))
        def _():
            pltpu.async_copy(ones_v, cnt_sh.at[dbufs[x_]], sem.at[12 + x_], add=True)

        @pl.when(j + 1 < NCHF)
        def _():
            wait_idx(j + 1, w)
            compute_gidx(w)
            start_gather(w)

        @pl.when(j >= 2)
        def _():
            pltpu.make_async_copy(rowss[z], acc_sh.at[dbufs[z]], sem.at[8 + z]).wait()

            @pl.when(cpred(j - 2))
            def _():
                pltpu.make_async_copy(ones_v, cnt_sh.at[dbufs[z]], sem.at[12 + z]).wait()

        @pl.when(j + 2 < NCHF)
        def _():
            load_idx(j + 2, z)

    load_idx(0, 0)
    wait_idx(0, 0)
    compute_gidx(0)
    start_gather(0)
    load_idx(1, 1)

    def body(jq, _):
        j = jq * 4
        for u in range(4):
            half(j + u, u)
        return 0
    lax.fori_loop(0, NCHF // 4, body, 0)

    for jj in (NCHF - 2, NCHF - 1):
        z = jj % 4
        pltpu.make_async_copy(rowss[z], acc_sh.at[dbufs[z]], sem.at[8 + z]).wait()

        @pl.when(cpred(jj))
        def _():
            pltpu.make_async_copy(ones_v, cnt_sh.at[dbufs[z]], sem.at[12 + z]).wait()

    rbase = pl.multiple_of(ebase + NCHF * CHUNK, 8)
    pltpu.sync_copy(src_hbm.at[pl.ds(rbase, REM)], srem)
    pltpu.sync_copy(dst_hbm.at[pl.ds(rbase, REM)], drem)
    for k in range(REM // 16):
        v = srem[pl.ds(k * 16, 16)]
        grem[pl.ds(k * 16, 16)] = v + v + c
    pltpu.sync_copy(xv_hbm.at[grem], rrem)
    pltpu.sync_copy(rrem, acc_sh.at[drem], add=True)

    @pl.when(c == 0)
    def _():
        pltpu.sync_copy(ones_v.at[pl.ds(0, REM)], cnt_sh.at[drem], add=True)

    plsc.subcore_barrier()

    for k in range(ROWS_PER_TILE // ZROWS):
        r = row0 + k * ZROWS
        pltpu.sync_copy(acc_sh.at[pl.ds(r, ZROWS)], zb)
        pltpu.sync_copy(zb, acc_out.at[c, pl.ds(r, ZROWS)])
    pltpu.sync_copy(cnt_sh.at[pl.ds(row0, ROWS_PER_TILE)], zb2)
    pltpu.sync_copy(zb2, cnt_out.at[c, pl.ds(row0, ROWS_PER_TILE)])


def _sc_wrap(xv_hbm, src_hbm, dst_hbm, acc_out, cnt_out,
             s0, s1, s2, s3, g0, g1, g2, g3, d0, d1, d2, d3,
             r0, r1, r2, r3, srem, grem, drem, rrem,
             ones_v, zb, zb2, sem, acc_sh, cnt_sh):
    _sc_aggregate_body(xv_hbm, src_hbm, dst_hbm, acc_out, cnt_out,
                       (s0, s1, s2, s3), (g0, g1, g2, g3), (d0, d1, d2, d3),
                       (r0, r1, r2, r3), srem, grem, drem, rrem,
                       ones_v, zb, zb2, sem, acc_sh, cnt_sh)


_sc_aggregate = functools.partial(
    pl.kernel,
    mesh=plsc.VectorSubcoreMesh(core_axis_name="c", subcore_axis_name="s"),
    compiler_params=pltpu.CompilerParams(use_tc_tiling_on_sc=False),
    out_type=[
        jax.ShapeDtypeStruct((NC, N_PAD, DH), jnp.float32),
        jax.ShapeDtypeStruct((NC, N_PAD, CNT_W), jnp.float32),
    ],
    scratch_types=(
        [pltpu.VMEM((CHUNK,), jnp.int32) for _ in range(12)]
        + [pltpu.VMEM((CHUNK, DH), jnp.float32) for _ in range(4)]
        + [pltpu.VMEM((REM,), jnp.int32) for _ in range(3)]
        + [
            pltpu.VMEM((REM, DH), jnp.float32),
            pltpu.VMEM((CHUNK, CNT_W), jnp.float32),
            pltpu.VMEM((ZROWS, DH), jnp.float32),
            pltpu.VMEM((ROWS_PER_TILE, CNT_W), jnp.float32),
            pltpu.SemaphoreType.DMA((16,)),
            pltpu.VMEM_SHARED((N_PAD, DH), jnp.float32),
            pltpu.VMEM_SHARED((N_PAD, CNT_W), jnp.float32),
        ]
    ),
)(_sc_wrap)


ROWS_PER_BLK = 1000


def _tc_finish_body(acc_ref, cnt_ref, w_ref, b_ref, out_ref):
    s = jnp.concatenate([acc_ref[0], acc_ref[1]], axis=1)
    cnt = cnt_ref[0] + cnt_ref[1]
    deg = cnt[:, 0:1]
    mean = s / jnp.maximum(deg, 1.0)
    h = lax.dot_general(mean, w_ref[...], (((1,), (1,)), ((), ())),
                        preferred_element_type=jnp.float32)
    out_ref[...] = h + b_ref[...]


def _tc_finish(acc, cnt, w, b):
    return pl.pallas_call(
        _tc_finish_body,
        grid=(N_NODES // ROWS_PER_BLK,),
        in_specs=[
            pl.BlockSpec((NC, ROWS_PER_BLK, DH), lambda i: (0, i, 0)),
            pl.BlockSpec((NC, ROWS_PER_BLK, CNT_W), lambda i: (0, i, 0)),
            pl.BlockSpec((D, D), lambda i: (0, 0)),
            pl.BlockSpec((1, D), lambda i: (0, 0)),
        ],
        out_specs=pl.BlockSpec((ROWS_PER_BLK, D), lambda i: (i, 0)),
        out_shape=jax.ShapeDtypeStruct((N_NODES, D), jnp.float32),
    )(acc, cnt, w, b)


@jax.jit
def kernel(x, edge_index, W, b):
    ei = edge_index.astype(jnp.int32)
    src = ei[0]
    dst = ei[1]
    xv = x.reshape(2 * N_NODES, DH)
    acc, cnt = _sc_aggregate(xv, src, dst)
    return _tc_finish(acc, cnt, W, b.reshape(1, D))


# speedup vs baseline: 13.4397x; 1.3318x over previous
import functools

import jax
import jax.numpy as jnp
from jax import lax
from jax.experimental import pallas as pl
from jax.experimental.pallas import tpu as pltpu
from jax.experimental.pallas import tpu_sc as plsc

N_NODES = 10000
N_EDGES = 320000
D = 128
DH = D // 2

NC = 2
NS = 16

E_PER_T = N_EDGES // NS
CHUNK = 128
NCHF = E_PER_T // CHUNK
NB = 6
REM = E_PER_T - NCHF * CHUNK
CHALF = NCHF // 2
N_PAD = 10240
ROWS_PER_TILE = N_PAD // NS
ZROWS = 128
CNT_W = 16


def _sc_aggregate_body(xv_hbm, ei_hbm, acc_out, cnt_out,
                       sbufs, gbufs, dbufs, rowss, srem, grem, drem, rrem,
                       ones_v, zb, zb2, sem, acc_sh, cnt_sh):
    c = lax.axis_index("c")
    s = lax.axis_index("s")
    ebase = s * E_PER_T

    def zrow(r, _):
        for k in range(DH // 16):
            zb[r, pl.ds(k * 16, 16)] = jnp.zeros((16,), jnp.float32)
        return 0
    lax.fori_loop(0, ZROWS, zrow, 0)

    def orow(r, _):
        ones_v[r, pl.ds(0, 16)] = jnp.ones((16,), jnp.float32)
        return 0
    lax.fori_loop(0, CHUNK, orow, 0)

    def z2row(r, _):
        zb2[r, pl.ds(0, 16)] = jnp.zeros((16,), jnp.float32)
        return 0
    lax.fori_loop(0, ROWS_PER_TILE, z2row, 0)

    row0 = s * ROWS_PER_TILE
    for k in range(ROWS_PER_TILE // ZROWS):
        pltpu.sync_copy(zb, acc_sh.at[pl.ds(row0 + k * ZROWS, ZROWS)])
    pltpu.sync_copy(zb2, cnt_sh.at[pl.ds(row0, ROWS_PER_TILE)])
    plsc.subcore_barrier()

    def cpred(j):
        return (j < CHALF) == (c == 0)

    def load_idx(j, slot):
        base = pl.multiple_of(ebase + j * CHUNK, CHUNK)
        pltpu.async_copy(ei_hbm.at[0, pl.ds(base, CHUNK)], sbufs[slot], sem.at[slot])
        pltpu.async_copy(ei_hbm.at[1, pl.ds(base, CHUNK)], dbufs[slot], sem.at[slot])

    def wait_idx(j, slot):
        base = pl.multiple_of(ebase + j * CHUNK, CHUNK)
        pltpu.make_async_copy(ei_hbm.at[0, pl.ds(base, CHUNK)], sbufs[slot], sem.at[slot]).wait()
        pltpu.make_async_copy(ei_hbm.at[1, pl.ds(base, CHUNK)], dbufs[slot], sem.at[slot]).wait()

    def compute_gidx(slot):
        for k in range(CHUNK // 16):
            v = sbufs[slot][pl.ds(k * 16, 16)]
            gbufs[slot][pl.ds(k * 16, 16)] = v + v + c

    def start_gather(slot):
        pltpu.async_copy(xv_hbm.at[gbufs[slot]], rowss[slot], sem.at[NB + slot])

    def wait_gather(slot):
        pltpu.make_async_copy(xv_hbm.at[gbufs[slot]], rowss[slot], sem.at[NB + slot]).wait()

    def wait_scatter(jj, z):
        pltpu.make_async_copy(rowss[z], acc_sh.at[dbufs[z]], sem.at[2 * NB + z]).wait()

        @pl.when(cpred(jj))
        def _():
            pltpu.make_async_copy(ones_v, cnt_sh.at[dbufs[z]], sem.at[3 * NB + z]).wait()

    def half(j, x_):
        wait_gather(x_)
        pltpu.async_copy(rowss[x_], acc_sh.at[dbufs[x_]], sem.at[2 * NB + x_], add=True)

        @pl.when(cpred(j))
        def _():
            pltpu.async_copy(ones_v, cnt_sh.at[dbufs[x_]], sem.at[3 * NB + x_], add=True)

        z = (x_ + 4) % NB

        @pl.when(j >= 2)
        def _():
            wait_scatter(j - 2, z)

        w2 = (x_ + 2) % NB

        @pl.when(j + 2 < NCHF)
        def _():
            wait_idx(j + 2, w2)
            compute_gidx(w2)
            start_gather(w2)

        @pl.when(j + 4 < NCHF)
        def _():
            load_idx(j + 4, z)

    for jj in range(4):
        load_idx(jj, jj)
    for jj in range(2):
        wait_idx(jj, jj)
        compute_gidx(jj)
        start_gather(jj)

    def body(jq, _):
        j = jq * NB
        for u in range(NB):
            half(j + u, u)
        return 0
    lax.fori_loop(0, NCHF // NB, body, 0)

    for jj in range((NCHF // NB) * NB, NCHF):
        half(jj, jj % NB)
    for jj in range(NCHF - 2, NCHF):
        wait_scatter(jj, jj % NB)

    rbase = pl.multiple_of(ebase + NCHF * CHUNK, 8)
    pltpu.sync_copy(ei_hbm.at[0, pl.ds(rbase, REM)], srem)
    pltpu.sync_copy(ei_hbm.at[1, pl.ds(rbase, REM)], drem)
    for k in range(REM // 16):
        v = srem[pl.ds(k * 16, 16)]
        grem[pl.ds(k * 16, 16)] = v + v + c
    pltpu.sync_copy(xv_hbm.at[grem], rrem)
    pltpu.sync_copy(rrem, acc_sh.at[drem], add=True)

    @pl.when(c == 0)
    def _():
        pltpu.sync_copy(ones_v.at[pl.ds(0, REM)], cnt_sh.at[drem], add=True)

    plsc.subcore_barrier()

    for k in range(ROWS_PER_TILE // ZROWS):
        r = row0 + k * ZROWS
        pltpu.sync_copy(acc_sh.at[pl.ds(r, ZROWS)], zb)
        pltpu.sync_copy(zb, acc_out.at[c, pl.ds(r, ZROWS)])
    pltpu.sync_copy(cnt_sh.at[pl.ds(row0, ROWS_PER_TILE)], zb2)
    pltpu.sync_copy(zb2, cnt_out.at[c, pl.ds(row0, ROWS_PER_TILE)])


def _sc_wrap(xv_hbm, ei_hbm, acc_out, cnt_out,
             s0, s1, s2, s3, s4, s5,
             g0, g1, g2, g3, g4, g5,
             d0, d1, d2, d3, d4, d5,
             r0, r1, r2, r3, r4, r5,
             srem, grem, drem, rrem,
             ones_v, zb, zb2, sem, acc_sh, cnt_sh):
    _sc_aggregate_body(xv_hbm, ei_hbm, acc_out, cnt_out,
                       (s0, s1, s2, s3, s4, s5),
                       (g0, g1, g2, g3, g4, g5),
                       (d0, d1, d2, d3, d4, d5),
                       (r0, r1, r2, r3, r4, r5),
                       srem, grem, drem, rrem,
                       ones_v, zb, zb2, sem, acc_sh, cnt_sh)


_sc_aggregate = functools.partial(
    pl.kernel,
    mesh=plsc.VectorSubcoreMesh(core_axis_name="c", subcore_axis_name="s"),
    compiler_params=pltpu.CompilerParams(use_tc_tiling_on_sc=False),
    out_type=[
        jax.ShapeDtypeStruct((NC, N_PAD, DH), jnp.float32),
        jax.ShapeDtypeStruct((NC, N_PAD, CNT_W), jnp.float32),
    ],
    scratch_types=(
        [pltpu.VMEM((CHUNK,), jnp.int32) for _ in range(18)]
        + [pltpu.VMEM((CHUNK, DH), jnp.float32) for _ in range(6)]
        + [pltpu.VMEM((REM,), jnp.int32) for _ in range(3)]
        + [
            pltpu.VMEM((REM, DH), jnp.float32),
            pltpu.VMEM((CHUNK, CNT_W), jnp.float32),
            pltpu.VMEM((ZROWS, DH), jnp.float32),
            pltpu.VMEM((ROWS_PER_TILE, CNT_W), jnp.float32),
            pltpu.SemaphoreType.DMA((24,)),
            pltpu.VMEM_SHARED((N_PAD, DH), jnp.float32),
            pltpu.VMEM_SHARED((N_PAD, CNT_W), jnp.float32),
        ]
    ),
)(_sc_wrap)


ROWS_PER_BLK = 1000


def _tc_finish_body(acc_ref, cnt_ref, w_ref, b_ref, out_ref):
    s = jnp.concatenate([acc_ref[0], acc_ref[1]], axis=1)
    cnt = cnt_ref[0] + cnt_ref[1]
    deg = cnt[:, 0:1]
    mean = s / jnp.maximum(deg, 1.0)
    h = lax.dot_general(mean, w_ref[...], (((1,), (1,)), ((), ())),
                        preferred_element_type=jnp.float32)
    out_ref[...] = h + b_ref[...]


def _tc_finish(acc, cnt, w, b):
    return pl.pallas_call(
        _tc_finish_body,
        grid=(N_NODES // ROWS_PER_BLK,),
        in_specs=[
            pl.BlockSpec((NC, ROWS_PER_BLK, DH), lambda i: (0, i, 0)),
            pl.BlockSpec((NC, ROWS_PER_BLK, CNT_W), lambda i: (0, i, 0)),
            pl.BlockSpec((D, D), lambda i: (0, 0)),
            pl.BlockSpec((1, D), lambda i: (0, 0)),
        ],
        out_specs=pl.BlockSpec((ROWS_PER_BLK, D), lambda i: (i, 0)),
        out_shape=jax.ShapeDtypeStruct((N_NODES, D), jnp.float32),
    )(acc, cnt, w, b)


@jax.jit
def kernel(x, edge_index, W, b):
    ei = edge_index.astype(jnp.int32)
    xv = x.reshape(2 * N_NODES, DH)
    acc, cnt = _sc_aggregate(xv, ei)
    return _tc_finish(acc, cnt, W, b.reshape(1, D))


# speedup vs baseline: 13.4977x; 1.0043x over previous
import functools

import jax
import jax.numpy as jnp
from jax import lax
from jax.experimental import pallas as pl
from jax.experimental.pallas import tpu as pltpu
from jax.experimental.pallas import tpu_sc as plsc

N_NODES = 10000
N_EDGES = 320000
D = 128
DH = D // 2

NC = 2
NS = 16

E_PER_T = N_EDGES // NS
CHUNK = 128
NCHF = E_PER_T // CHUNK
NB = 6
REM = E_PER_T - NCHF * CHUNK
CHALF = NCHF // 2
N_PAD = 10240
ROWS_PER_TILE = N_PAD // NS
ZROWS = 128
CNT_W = 16


def _sc_aggregate_body(xv_hbm, ei_hbm, acc_out, cnt_out,
                       sbufs, gbufs, dbufs, rowss, srem, grem, drem, rrem,
                       ones_v, zb, zb2, sem, acc_sh, cnt_sh):
    c = lax.axis_index("c")
    s = lax.axis_index("s")
    ebase = s * E_PER_T

    def zrow(r, _):
        for k in range(DH // 16):
            zb[r, pl.ds(k * 16, 16)] = jnp.zeros((16,), jnp.float32)
        return 0
    lax.fori_loop(0, ZROWS, zrow, 0)

    def orow(r, _):
        ones_v[r, pl.ds(0, 16)] = jnp.ones((16,), jnp.float32)
        return 0
    lax.fori_loop(0, CHUNK, orow, 0)

    def z2row(r, _):
        zb2[r, pl.ds(0, 16)] = jnp.zeros((16,), jnp.float32)
        return 0
    lax.fori_loop(0, ROWS_PER_TILE, z2row, 0)

    row0 = s * ROWS_PER_TILE
    for k in range(ROWS_PER_TILE // ZROWS):
        pltpu.async_copy(zb, acc_sh.at[pl.ds(row0 + k * ZROWS, ZROWS)],
                         sem.at[4 * NB])
    pltpu.async_copy(zb2, cnt_sh.at[pl.ds(row0, ROWS_PER_TILE)], sem.at[4 * NB])
    for k in range(ROWS_PER_TILE // ZROWS):
        pltpu.make_async_copy(zb, acc_sh.at[pl.ds(row0 + k * ZROWS, ZROWS)],
                              sem.at[4 * NB]).wait()
    pltpu.make_async_copy(zb2, cnt_sh.at[pl.ds(row0, ROWS_PER_TILE)],
                          sem.at[4 * NB]).wait()
    plsc.subcore_barrier()

    def cpred(j):
        return (j < CHALF) == (c == 0)

    def load_idx(j, slot):
        base = pl.multiple_of(ebase + j * CHUNK, CHUNK)
        pltpu.async_copy(ei_hbm.at[0, pl.ds(base, CHUNK)], sbufs[slot], sem.at[slot])
        pltpu.async_copy(ei_hbm.at[1, pl.ds(base, CHUNK)], dbufs[slot], sem.at[slot])

    def wait_idx(j, slot):
        base = pl.multiple_of(ebase + j * CHUNK, CHUNK)
        pltpu.make_async_copy(ei_hbm.at[0, pl.ds(base, CHUNK)], sbufs[slot], sem.at[slot]).wait()
        pltpu.make_async_copy(ei_hbm.at[1, pl.ds(base, CHUNK)], dbufs[slot], sem.at[slot]).wait()

    def compute_gidx(slot):
        for k in range(CHUNK // 16):
            v = sbufs[slot][pl.ds(k * 16, 16)]
            gbufs[slot][pl.ds(k * 16, 16)] = v + v + c

    def start_gather(slot):
        pltpu.async_copy(xv_hbm.at[gbufs[slot]], rowss[slot], sem.at[NB + slot])

    def wait_gather(slot):
        pltpu.make_async_copy(xv_hbm.at[gbufs[slot]], rowss[slot], sem.at[NB + slot]).wait()

    def wait_scatter(jj, z):
        pltpu.make_async_copy(rowss[z], acc_sh.at[dbufs[z]], sem.at[2 * NB + z]).wait()

        @pl.when(cpred(jj))
        def _():
            pltpu.make_async_copy(ones_v, cnt_sh.at[dbufs[z]], sem.at[3 * NB + z]).wait()

    def half(j, x_):
        wait_gather(x_)
        pltpu.async_copy(rowss[x_], acc_sh.at[dbufs[x_]], sem.at[2 * NB + x_], add=True)

        @pl.when(cpred(j))
        def _():
            pltpu.async_copy(ones_v, cnt_sh.at[dbufs[x_]], sem.at[3 * NB + x_], add=True)

        z = (x_ + 4) % NB

        @pl.when(j >= 2)
        def _():
            wait_scatter(j - 2, z)

        w2 = (x_ + 2) % NB

        @pl.when(j + 2 < NCHF)
        def _():
            wait_idx(j + 2, w2)
            compute_gidx(w2)
            start_gather(w2)

        @pl.when(j + 4 < NCHF)
        def _():
            load_idx(j + 4, z)

    for jj in range(4):
        load_idx(jj, jj)
    for jj in range(2):
        wait_idx(jj, jj)
        compute_gidx(jj)
        start_gather(jj)

    def body(jq, _):
        j = jq * NB
        for u in range(NB):
            half(j + u, u)
        return 0
    lax.fori_loop(0, NCHF // NB, body, 0)

    for jj in range((NCHF // NB) * NB, NCHF):
        half(jj, jj % NB)
    for jj in range(NCHF - 2, NCHF):
        wait_scatter(jj, jj % NB)

    rbase = pl.multiple_of(ebase + NCHF * CHUNK, 8)
    pltpu.sync_copy(ei_hbm.at[0, pl.ds(rbase, REM)], srem)
    pltpu.sync_copy(ei_hbm.at[1, pl.ds(rbase, REM)], drem)
    for k in range(REM // 16):
        v = srem[pl.ds(k * 16, 16)]
        grem[pl.ds(k * 16, 16)] = v + v + c
    pltpu.sync_copy(xv_hbm.at[grem], rrem)
    pltpu.sync_copy(rrem, acc_sh.at[drem], add=True)

    @pl.when(c == 0)
    def _():
        pltpu.sync_copy(ones_v.at[pl.ds(0, REM)], cnt_sh.at[drem], add=True)

    plsc.subcore_barrier()

    pltpu.async_copy(acc_sh.at[pl.ds(row0, ROWS_PER_TILE)],
                     acc_out.at[c, pl.ds(row0, ROWS_PER_TILE)], sem.at[4 * NB])
    pltpu.async_copy(cnt_sh.at[pl.ds(row0, ROWS_PER_TILE)],
                     cnt_out.at[c, pl.ds(row0, ROWS_PER_TILE)], sem.at[4 * NB])
    pltpu.make_async_copy(acc_sh.at[pl.ds(row0, ROWS_PER_TILE)],
                          acc_out.at[c, pl.ds(row0, ROWS_PER_TILE)],
                          sem.at[4 * NB]).wait()
    pltpu.make_async_copy(cnt_sh.at[pl.ds(row0, ROWS_PER_TILE)],
                          cnt_out.at[c, pl.ds(row0, ROWS_PER_TILE)],
                          sem.at[4 * NB]).wait()


def _sc_wrap(xv_hbm, ei_hbm, acc_out, cnt_out,
             s0, s1, s2, s3, s4, s5,
             g0, g1, g2, g3, g4, g5,
             d0, d1, d2, d3, d4, d5,
             r0, r1, r2, r3, r4, r5,
             srem, grem, drem, rrem,
             ones_v, zb, zb2, sem, acc_sh, cnt_sh):
    _sc_aggregate_body(xv_hbm, ei_hbm, acc_out, cnt_out,
                       (s0, s1, s2, s3, s4, s5),
                       (g0, g1, g2, g3, g4, g5),
                       (d0, d1, d2, d3, d4, d5),
                       (r0, r1, r2, r3, r4, r5),
                       srem, grem, drem, rrem,
                       ones_v, zb, zb2, sem, acc_sh, cnt_sh)


_sc_aggregate = functools.partial(
    pl.kernel,
    mesh=plsc.VectorSubcoreMesh(core_axis_name="c", subcore_axis_name="s"),
    compiler_params=pltpu.CompilerParams(use_tc_tiling_on_sc=False),
    out_type=[
        jax.ShapeDtypeStruct((NC, N_PAD, DH), jnp.float32),
        jax.ShapeDtypeStruct((NC, N_PAD, CNT_W), jnp.float32),
    ],
    scratch_types=(
        [pltpu.VMEM((CHUNK,), jnp.int32) for _ in range(18)]
        + [pltpu.VMEM((CHUNK, DH), jnp.float32) for _ in range(6)]
        + [pltpu.VMEM((REM,), jnp.int32) for _ in range(3)]
        + [
            pltpu.VMEM((REM, DH), jnp.float32),
            pltpu.VMEM((CHUNK, CNT_W), jnp.float32),
            pltpu.VMEM((ZROWS, DH), jnp.float32),
            pltpu.VMEM((ROWS_PER_TILE, CNT_W), jnp.float32),
            pltpu.SemaphoreType.DMA((25,)),
            pltpu.VMEM_SHARED((N_PAD, DH), jnp.float32),
            pltpu.VMEM_SHARED((N_PAD, CNT_W), jnp.float32),
        ]
    ),
)(_sc_wrap)


ROWS_PER_BLK = 1000


def _tc_finish_body(acc_ref, cnt_ref, w_ref, b_ref, out_ref):
    s = jnp.concatenate([acc_ref[0], acc_ref[1]], axis=1)
    cnt = cnt_ref[0] + cnt_ref[1]
    deg = cnt[:, 0:1]
    mean = s / jnp.maximum(deg, 1.0)
    h = lax.dot_general(mean, w_ref[...], (((1,), (1,)), ((), ())),
                        preferred_element_type=jnp.float32)
    out_ref[...] = h + b_ref[...]


def _tc_finish(acc, cnt, w, b):
    return pl.pallas_call(
        _tc_finish_body,
        grid=(N_NODES // ROWS_PER_BLK,),
        in_specs=[
            pl.BlockSpec((NC, ROWS_PER_BLK, DH), lambda i: (0, i, 0)),
            pl.BlockSpec((NC, ROWS_PER_BLK, CNT_W), lambda i: (0, i, 0)),
            pl.BlockSpec((D, D), lambda i: (0, 0)),
            pl.BlockSpec((1, D), lambda i: (0, 0)),
        ],
        out_specs=pl.BlockSpec((ROWS_PER_BLK, D), lambda i: (i, 0)),
        out_shape=jax.ShapeDtypeStruct((N_NODES, D), jnp.float32),
    )(acc, cnt, w, b)


@jax.jit
def kernel(x, edge_index, W, b):
    ei = edge_index.astype(jnp.int32)
    xv = x.reshape(2 * N_NODES, DH)
    acc, cnt = _sc_aggregate(xv, ei)
    return _tc_finish(acc, cnt, W, b.reshape(1, D))


# speedup vs baseline: 13.5204x; 1.0017x over previous
import functools

import jax
import jax.numpy as jnp
from jax import lax
from jax.experimental import pallas as pl
from jax.experimental.pallas import tpu as pltpu
from jax.experimental.pallas import tpu_sc as plsc

N_NODES = 10000
N_EDGES = 320000
D = 128
DH = D // 2

NC = 2
NS = 16

E_PER_T = N_EDGES // NS
CHUNK = 128
NCHF = E_PER_T // CHUNK
NB = 6
REM = E_PER_T - NCHF * CHUNK
CHALF = NCHF // 2
N_PAD = 10240
ROWS_PER_TILE = N_PAD // NS
ZROWS = 128
CNT_W = 8


def _sc_aggregate_body(xv_hbm, ei_hbm, acc_out, cnt_out,
                       sbufs, gbufs, dbufs, rowss, srem, grem, drem, rrem,
                       ones_v, zb, zb2, sem, acc_sh, cnt_sh):
    c = lax.axis_index("c")
    s = lax.axis_index("s")
    ebase = s * E_PER_T

    def zrow(r, _):
        for k in range(DH // 16):
            zb[r, pl.ds(k * 16, 16)] = jnp.zeros((16,), jnp.float32)
        return 0
    lax.fori_loop(0, ZROWS, zrow, 0)

    def orow(r, _):
        ones_v[r, pl.ds(0, 16)] = jnp.ones((16,), jnp.float32)
        return 0
    lax.fori_loop(0, CHUNK, orow, 0)

    def z2row(r, _):
        zb2[r, pl.ds(0, 16)] = jnp.zeros((16,), jnp.float32)
        return 0
    lax.fori_loop(0, ROWS_PER_TILE, z2row, 0)

    row0 = s * ROWS_PER_TILE
    for k in range(ROWS_PER_TILE // ZROWS):
        pltpu.async_copy(zb, acc_sh.at[pl.ds(row0 + k * ZROWS, ZROWS)],
                         sem.at[4 * NB])
    pltpu.async_copy(zb2, cnt_sh.at[pl.ds(row0, ROWS_PER_TILE)], sem.at[4 * NB])
    for k in range(ROWS_PER_TILE // ZROWS):
        pltpu.make_async_copy(zb, acc_sh.at[pl.ds(row0 + k * ZROWS, ZROWS)],
                              sem.at[4 * NB]).wait()
    pltpu.make_async_copy(zb2, cnt_sh.at[pl.ds(row0, ROWS_PER_TILE)],
                          sem.at[4 * NB]).wait()
    plsc.subcore_barrier()

    def cpred(j):
        return (j < CHALF) == (c == 0)

    def load_idx(j, slot):
        base = pl.multiple_of(ebase + j * CHUNK, CHUNK)
        pltpu.async_copy(ei_hbm.at[0, pl.ds(base, CHUNK)], sbufs[slot], sem.at[slot])
        pltpu.async_copy(ei_hbm.at[1, pl.ds(base, CHUNK)], dbufs[slot], sem.at[slot])

    def wait_idx(j, slot):
        base = pl.multiple_of(ebase + j * CHUNK, CHUNK)
        pltpu.make_async_copy(ei_hbm.at[0, pl.ds(base, CHUNK)], sbufs[slot], sem.at[slot]).wait()
        pltpu.make_async_copy(ei_hbm.at[1, pl.ds(base, CHUNK)], dbufs[slot], sem.at[slot]).wait()

    def compute_gidx(slot):
        for k in range(CHUNK // 16):
            v = sbufs[slot][pl.ds(k * 16, 16)]
            gbufs[slot][pl.ds(k * 16, 16)] = v + v + c

    def start_gather(slot):
        pltpu.async_copy(xv_hbm.at[gbufs[slot]], rowss[slot], sem.at[NB + slot])

    def wait_gather(slot):
        pltpu.make_async_copy(xv_hbm.at[gbufs[slot]], rowss[slot], sem.at[NB + slot]).wait()

    def wait_scatter(jj, z):
        pltpu.make_async_copy(rowss[z], acc_sh.at[dbufs[z]], sem.at[2 * NB + z]).wait()

        @pl.when(cpred(jj))
        def _():
            pltpu.make_async_copy(ones_v, cnt_sh.at[dbufs[z]], sem.at[3 * NB + z]).wait()

    def half(j, x_):
        wait_gather(x_)
        pltpu.async_copy(rowss[x_], acc_sh.at[dbufs[x_]], sem.at[2 * NB + x_], add=True)

        @pl.when(cpred(j))
        def _():
            pltpu.async_copy(ones_v, cnt_sh.at[dbufs[x_]], sem.at[3 * NB + x_], add=True)

        z = (x_ + 4) % NB

        @pl.when(j >= 2)
        def _():
            wait_scatter(j - 2, z)

        w2 = (x_ + 2) % NB

        @pl.when(j + 2 < NCHF)
        def _():
            wait_idx(j + 2, w2)
            compute_gidx(w2)
            start_gather(w2)

        @pl.when(j + 4 < NCHF)
        def _():
            load_idx(j + 4, z)

    for jj in range(4):
        load_idx(jj, jj)
    for jj in range(2):
        wait_idx(jj, jj)
        compute_gidx(jj)
        start_gather(jj)

    def body(jq, _):
        j = jq * NB
        for u in range(NB):
            half(j + u, u)
        return 0
    lax.fori_loop(0, NCHF // NB, body, 0)

    for jj in range((NCHF // NB) * NB, NCHF):
        half(jj, jj % NB)
    for jj in range(NCHF - 2, NCHF):
        wait_scatter(jj, jj % NB)

    rbase = pl.multiple_of(ebase + NCHF * CHUNK, 8)
    pltpu.sync_copy(ei_hbm.at[0, pl.ds(rbase, REM)], srem)
    pltpu.sync_copy(ei_hbm.at[1, pl.ds(rbase, REM)], drem)
    for k in range(REM // 16):
        v = srem[pl.ds(k * 16, 16)]
        grem[pl.ds(k * 16, 16)] = v + v + c
    pltpu.sync_copy(xv_hbm.at[grem], rrem)
    pltpu.sync_copy(rrem, acc_sh.at[drem], add=True)

    @pl.when(c == 0)
    def _():
        pltpu.sync_copy(ones_v.at[pl.ds(0, REM)], cnt_sh.at[drem], add=True)

    plsc.subcore_barrier()

    pltpu.async_copy(acc_sh.at[pl.ds(row0, ROWS_PER_TILE)],
                     acc_out.at[c, pl.ds(row0, ROWS_PER_TILE)], sem.at[4 * NB])
    pltpu.async_copy(cnt_sh.at[pl.ds(row0, ROWS_PER_TILE)],
                     cnt_out.at[c, pl.ds(row0, ROWS_PER_TILE)], sem.at[4 * NB])
    pltpu.make_async_copy(acc_sh.at[pl.ds(row0, ROWS_PER_TILE)],
                          acc_out.at[c, pl.ds(row0, ROWS_PER_TILE)],
                          sem.at[4 * NB]).wait()
    pltpu.make_async_copy(cnt_sh.at[pl.ds(row0, ROWS_PER_TILE)],
                          cnt_out.at[c, pl.ds(row0, ROWS_PER_TILE)],
                          sem.at[4 * NB]).wait()


def _sc_wrap(xv_hbm, ei_hbm, acc_out, cnt_out,
             s0, s1, s2, s3, s4, s5,
             g0, g1, g2, g3, g4, g5,
             d0, d1, d2, d3, d4, d5,
             r0, r1, r2, r3, r4, r5,
             srem, grem, drem, rrem,
             ones_v, zb, zb2, sem, acc_sh, cnt_sh):
    _sc_aggregate_body(xv_hbm, ei_hbm, acc_out, cnt_out,
                       (s0, s1, s2, s3, s4, s5),
                       (g0, g1, g2, g3, g4, g5),
                       (d0, d1, d2, d3, d4, d5),
                       (r0, r1, r2, r3, r4, r5),
                       srem, grem, drem, rrem,
                       ones_v, zb, zb2, sem, acc_sh, cnt_sh)


_sc_aggregate = functools.partial(
    pl.kernel,
    mesh=plsc.VectorSubcoreMesh(core_axis_name="c", subcore_axis_name="s"),
    compiler_params=pltpu.CompilerParams(use_tc_tiling_on_sc=False),
    out_type=[
        jax.ShapeDtypeStruct((NC, N_PAD, DH), jnp.float32),
        jax.ShapeDtypeStruct((NC, N_PAD, CNT_W), jnp.float32),
    ],
    scratch_types=(
        [pltpu.VMEM((CHUNK,), jnp.int32) for _ in range(18)]
        + [pltpu.VMEM((CHUNK, DH), jnp.float32) for _ in range(6)]
        + [pltpu.VMEM((REM,), jnp.int32) for _ in range(3)]
        + [
            pltpu.VMEM((REM, DH), jnp.float32),
            pltpu.VMEM((CHUNK, CNT_W), jnp.float32),
            pltpu.VMEM((ZROWS, DH), jnp.float32),
            pltpu.VMEM((ROWS_PER_TILE, CNT_W), jnp.float32),
            pltpu.SemaphoreType.DMA((25,)),
            pltpu.VMEM_SHARED((N_PAD, DH), jnp.float32),
            pltpu.VMEM_SHARED((N_PAD, CNT_W), jnp.float32),
        ]
    ),
)(_sc_wrap)


ROWS_PER_BLK = 1000


def _tc_finish_body(acc_ref, cnt_ref, w_ref, b_ref, out_ref):
    s = jnp.concatenate([acc_ref[0], acc_ref[1]], axis=1)
    cnt = cnt_ref[0] + cnt_ref[1]
    deg = cnt[:, 0:1]
    mean = s / jnp.maximum(deg, 1.0)
    h = lax.dot_general(mean, w_ref[...], (((1,), (1,)), ((), ())),
                        preferred_element_type=jnp.float32)
    out_ref[...] = h + b_ref[...]


def _tc_finish(acc, cnt, w, b):
    return pl.pallas_call(
        _tc_finish_body,
        grid=(N_NODES // ROWS_PER_BLK,),
        in_specs=[
            pl.BlockSpec((NC, ROWS_PER_BLK, DH), lambda i: (0, i, 0)),
            pl.BlockSpec((NC, ROWS_PER_BLK, CNT_W), lambda i: (0, i, 0)),
            pl.BlockSpec((D, D), lambda i: (0, 0)),
            pl.BlockSpec((1, D), lambda i: (0, 0)),
        ],
        out_specs=pl.BlockSpec((ROWS_PER_BLK, D), lambda i: (i, 0)),
        out_shape=jax.ShapeDtypeStruct((N_NODES, D), jnp.float32),
    )(acc, cnt, w, b)


@jax.jit
def kernel(x, edge_index, W, b):
    ei = edge_index.astype(jnp.int32)
    xv = x.reshape(2 * N_NODES, DH)
    acc, cnt = _sc_aggregate(xv, ei)
    return _tc_finish(acc, cnt, W, b.reshape(1, D))


# speedup vs baseline: 14.6022x; 1.0800x over previous
import functools

import jax
import jax.numpy as jnp
from jax import lax
from jax.experimental import pallas as pl
from jax.experimental.pallas import tpu as pltpu
from jax.experimental.pallas import tpu_sc as plsc

N_NODES = 10000
N_EDGES = 320000
D = 128
DH = D // 2

NC = 2
NS = 16

E_PER_T = N_EDGES // NS
CHUNK = 128
NCHF = E_PER_T // CHUNK
NB = 6
REM = E_PER_T - NCHF * CHUNK
CHALF = NCHF // 2
N_PAD = 10240
ROWS_PER_TILE = N_PAD // NS
ZROWS = 128
CNT_W = 16


def _sc_aggregate_body(xv_hbm, ei_hbm, acc_out, cnt_out,
                       sbufs, gbufs, dbufs, rowss, srem, grem, drem, rrem,
                       ones_v, zb, zb2, sem, acc_sh, cnt_sh):
    c = lax.axis_index("c")
    s = lax.axis_index("s")
    ebase = s * E_PER_T

    def zrow(r, _):
        for k in range(DH // 16):
            zb[r, pl.ds(k * 16, 16)] = jnp.zeros((16,), jnp.float32)
        return 0
    lax.fori_loop(0, ZROWS, zrow, 0)

    def orow(r, _):
        ones_v[r, pl.ds(0, 16)] = jnp.ones((16,), jnp.float32)
        return 0
    lax.fori_loop(0, CHUNK, orow, 0)

    def z2row(r, _):
        zb2[r, pl.ds(0, 16)] = jnp.zeros((16,), jnp.float32)
        return 0
    lax.fori_loop(0, ROWS_PER_TILE, z2row, 0)

    row0 = s * ROWS_PER_TILE
    for k in range(ROWS_PER_TILE // ZROWS):
        pltpu.async_copy(zb, acc_sh.at[pl.ds(row0 + k * ZROWS, ZROWS)],
                         sem.at[4 * NB])
    pltpu.async_copy(zb2, cnt_sh.at[pl.ds(row0, ROWS_PER_TILE)], sem.at[4 * NB])
    for k in range(ROWS_PER_TILE // ZROWS):
        pltpu.make_async_copy(zb, acc_sh.at[pl.ds(row0 + k * ZROWS, ZROWS)],
                              sem.at[4 * NB]).wait()
    pltpu.make_async_copy(zb2, cnt_sh.at[pl.ds(row0, ROWS_PER_TILE)],
                          sem.at[4 * NB]).wait()
    plsc.subcore_barrier()

    def cpred(j):
        return (j < CHALF) == (c == 0)

    def load_idx(j, slot):
        base = pl.multiple_of(ebase + j * CHUNK, CHUNK)
        pltpu.async_copy(ei_hbm.at[0, pl.ds(base, CHUNK)], sbufs[slot], sem.at[slot])
        pltpu.async_copy(ei_hbm.at[1, pl.ds(base, CHUNK)], dbufs[slot], sem.at[slot])

    def wait_idx(j, slot):
        base = pl.multiple_of(ebase + j * CHUNK, CHUNK)
        pltpu.make_async_copy(ei_hbm.at[0, pl.ds(base, CHUNK)], sbufs[slot], sem.at[slot]).wait()
        pltpu.make_async_copy(ei_hbm.at[1, pl.ds(base, CHUNK)], dbufs[slot], sem.at[slot]).wait()

    def compute_gidx(slot):
        for k in range(CHUNK // 16):
            v = sbufs[slot][pl.ds(k * 16, 16)]
            gbufs[slot][pl.ds(k * 16, 16)] = v + v + c

    def start_gather(slot):
        pltpu.async_copy(xv_hbm.at[gbufs[slot]], rowss[slot], sem.at[NB + slot])

    def wait_gather(slot):
        pltpu.make_async_copy(xv_hbm.at[gbufs[slot]], rowss[slot], sem.at[NB + slot]).wait()

    def wait_scatter(jj, z):
        pltpu.make_async_copy(rowss[z], acc_sh.at[dbufs[z]], sem.at[2 * NB + z]).wait()

        @pl.when(cpred(jj))
        def _():
            pltpu.make_async_copy(ones_v, cnt_sh.at[dbufs[z]], sem.at[3 * NB + z]).wait()

    def half(j, x_):
        wait_gather(x_)
        pltpu.async_copy(rowss[x_], acc_sh.at[dbufs[x_]], sem.at[2 * NB + x_], add=True)

        @pl.when(cpred(j))
        def _():
            pltpu.async_copy(ones_v, cnt_sh.at[dbufs[x_]], sem.at[3 * NB + x_], add=True)

        z = (x_ + 4) % NB

        @pl.when(j >= 2)
        def _():
            wait_scatter(j - 2, z)

        w3 = (x_ + 3) % NB

        @pl.when(j + 3 < NCHF)
        def _():
            wait_idx(j + 3, w3)
            compute_gidx(w3)
            start_gather(w3)

        @pl.when(j + 4 < NCHF)
        def _():
            load_idx(j + 4, z)

    for jj in range(4):
        load_idx(jj, jj)
    for jj in range(3):
        wait_idx(jj, jj)
        compute_gidx(jj)
        start_gather(jj)

    def body(jq, _):
        j = jq * NB
        for u in range(NB):
            half(j + u, u)
        return 0
    lax.fori_loop(0, NCHF // NB, body, 0)

    for jj in range((NCHF // NB) * NB, NCHF):
        half(jj, jj % NB)
    for jj in range(NCHF - 2, NCHF):
        wait_scatter(jj, jj % NB)

    rbase = pl.multiple_of(ebase + NCHF * CHUNK, 8)
    pltpu.sync_copy(ei_hbm.at[0, pl.ds(rbase, REM)], srem)
    pltpu.sync_copy(ei_hbm.at[1, pl.ds(rbase, REM)], drem)
    for k in range(REM // 16):
        v = srem[pl.ds(k * 16, 16)]
        grem[pl.ds(k * 16, 16)] = v + v + c
    pltpu.sync_copy(xv_hbm.at[grem], rrem)
    pltpu.sync_copy(rrem, acc_sh.at[drem], add=True)

    @pl.when(c == 0)
    def _():
        pltpu.sync_copy(ones_v.at[pl.ds(0, REM)], cnt_sh.at[drem], add=True)

    plsc.subcore_barrier()

    pltpu.async_copy(acc_sh.at[pl.ds(row0, ROWS_PER_TILE)],
                     acc_out.at[c, pl.ds(row0, ROWS_PER_TILE)], sem.at[4 * NB])
    pltpu.async_copy(cnt_sh.at[pl.ds(row0, ROWS_PER_TILE)],
                     cnt_out.at[c, pl.ds(row0, ROWS_PER_TILE)], sem.at[4 * NB])
    pltpu.make_async_copy(acc_sh.at[pl.ds(row0, ROWS_PER_TILE)],
                          acc_out.at[c, pl.ds(row0, ROWS_PER_TILE)],
                          sem.at[4 * NB]).wait()
    pltpu.make_async_copy(cnt_sh.at[pl.ds(row0, ROWS_PER_TILE)],
                          cnt_out.at[c, pl.ds(row0, ROWS_PER_TILE)],
                          sem.at[4 * NB]).wait()


def _sc_wrap(xv_hbm, ei_hbm, acc_out, cnt_out,
             s0, s1, s2, s3, s4, s5,
             g0, g1, g2, g3, g4, g5,
             d0, d1, d2, d3, d4, d5,
             r0, r1, r2, r3, r4, r5,
             srem, grem, drem, rrem,
             ones_v, zb, zb2, sem, acc_sh, cnt_sh):
    _sc_aggregate_body(xv_hbm, ei_hbm, acc_out, cnt_out,
                       (s0, s1, s2, s3, s4, s5),
                       (g0, g1, g2, g3, g4, g5),
                       (d0, d1, d2, d3, d4, d5),
                       (r0, r1, r2, r3, r4, r5),
                       srem, grem, drem, rrem,
                       ones_v, zb, zb2, sem, acc_sh, cnt_sh)


_sc_aggregate = functools.partial(
    pl.kernel,
    mesh=plsc.VectorSubcoreMesh(core_axis_name="c", subcore_axis_name="s"),
    compiler_params=pltpu.CompilerParams(use_tc_tiling_on_sc=False),
    out_type=[
        jax.ShapeDtypeStruct((NC, N_PAD, DH), jnp.float32),
        jax.ShapeDtypeStruct((NC, N_PAD, CNT_W), jnp.float32),
    ],
    scratch_types=(
        [pltpu.VMEM((CHUNK,), jnp.int32) for _ in range(18)]
        + [pltpu.VMEM((CHUNK, DH), jnp.float32) for _ in range(6)]
        + [pltpu.VMEM((REM,), jnp.int32) for _ in range(3)]
        + [
            pltpu.VMEM((REM, DH), jnp.float32),
            pltpu.VMEM((CHUNK, CNT_W), jnp.float32),
            pltpu.VMEM((ZROWS, DH), jnp.float32),
            pltpu.VMEM((ROWS_PER_TILE, CNT_W), jnp.float32),
            pltpu.SemaphoreType.DMA((25,)),
            pltpu.VMEM_SHARED((N_PAD, DH), jnp.float32),
            pltpu.VMEM_SHARED((N_PAD, CNT_W), jnp.float32),
        ]
    ),
)(_sc_wrap)


ROWS_PER_BLK = 1000


def _tc_finish_body(acc_ref, cnt_ref, w_ref, b_ref, out_ref):
    s = jnp.concatenate([acc_ref[0], acc_ref[1]], axis=1)
    cnt = cnt_ref[0] + cnt_ref[1]
    deg = cnt[:, 0:1]
    mean = s / jnp.maximum(deg, 1.0)
    h = lax.dot_general(mean, w_ref[...], (((1,), (1,)), ((), ())),
                        preferred_element_type=jnp.float32)
    out_ref[...] = h + b_ref[...]


def _tc_finish(acc, cnt, w, b):
    return pl.pallas_call(
        _tc_finish_body,
        grid=(N_NODES // ROWS_PER_BLK,),
        in_specs=[
            pl.BlockSpec((NC, ROWS_PER_BLK, DH), lambda i: (0, i, 0)),
            pl.BlockSpec((NC, ROWS_PER_BLK, CNT_W), lambda i: (0, i, 0)),
            pl.BlockSpec((D, D), lambda i: (0, 0)),
            pl.BlockSpec((1, D), lambda i: (0, 0)),
        ],
        out_specs=pl.BlockSpec((ROWS_PER_BLK, D), lambda i: (i, 0)),
        out_shape=jax.ShapeDtypeStruct((N_NODES, D), jnp.float32),
    )(acc, cnt, w, b)


@jax.jit
def kernel(x, edge_index, W, b):
    ei = edge_index.astype(jnp.int32)
    xv = x.reshape(2 * N_NODES, DH)
    acc, cnt = _sc_aggregate(xv, ei)
    return _tc_finish(acc, cnt, W, b.reshape(1, D))


# speedup vs baseline: 15.2287x; 1.0429x over previous
import functools

import jax
import jax.numpy as jnp
from jax import lax
from jax.experimental import pallas as pl
from jax.experimental.pallas import tpu as pltpu
from jax.experimental.pallas import tpu_sc as plsc

N_NODES = 10000
N_EDGES = 320000
D = 128
DH = D // 2

NC = 2
NS = 16

E_PER_T = N_EDGES // NS
CHUNK = 128
NCHF = E_PER_T // CHUNK
NIB = 8
NRB = 6
UNROLL = 24
REM = E_PER_T - NCHF * CHUNK
CHALF = NCHF // 2
N_PAD = 10240
ROWS_PER_TILE = N_PAD // NS
ZROWS = 128
CNT_W = 16


def _sc_aggregate_body(xv_hbm, ei_hbm, acc_out, cnt_out,
                       sbufs, gbufs, dbufs, rowss, srem, grem, drem, rrem,
                       ones_v, zb, zb2, sem, acc_sh, cnt_sh):
    c = lax.axis_index("c")
    s = lax.axis_index("s")
    ebase = s * E_PER_T

    def zrow(r, _):
        for k in range(DH // 16):
            zb[r, pl.ds(k * 16, 16)] = jnp.zeros((16,), jnp.float32)
        return 0
    lax.fori_loop(0, ZROWS, zrow, 0)

    def orow(r, _):
        ones_v[r, pl.ds(0, 16)] = jnp.ones((16,), jnp.float32)
        return 0
    lax.fori_loop(0, CHUNK, orow, 0)

    def z2row(r, _):
        zb2[r, pl.ds(0, 16)] = jnp.zeros((16,), jnp.float32)
        return 0
    lax.fori_loop(0, ROWS_PER_TILE, z2row, 0)

    row0 = s * ROWS_PER_TILE
    for k in range(ROWS_PER_TILE // ZROWS):
        pltpu.async_copy(zb, acc_sh.at[pl.ds(row0 + k * ZROWS, ZROWS)],
                         sem.at[NIB + 3 * NRB])
    pltpu.async_copy(zb2, cnt_sh.at[pl.ds(row0, ROWS_PER_TILE)], sem.at[NIB + 3 * NRB])
    for k in range(ROWS_PER_TILE // ZROWS):
        pltpu.make_async_copy(zb, acc_sh.at[pl.ds(row0 + k * ZROWS, ZROWS)],
                              sem.at[NIB + 3 * NRB]).wait()
    pltpu.make_async_copy(zb2, cnt_sh.at[pl.ds(row0, ROWS_PER_TILE)],
                          sem.at[NIB + 3 * NRB]).wait()
    plsc.subcore_barrier()

    def cpred(j):
        return (j < CHALF) == (c == 0)

    def load_idx(j, slot):
        base = pl.multiple_of(ebase + j * CHUNK, CHUNK)
        pltpu.async_copy(ei_hbm.at[0, pl.ds(base, CHUNK)], sbufs[slot], sem.at[slot])
        pltpu.async_copy(ei_hbm.at[1, pl.ds(base, CHUNK)], dbufs[slot], sem.at[slot])

    def wait_idx(j, slot):
        base = pl.multiple_of(ebase + j * CHUNK, CHUNK)
        pltpu.make_async_copy(ei_hbm.at[0, pl.ds(base, CHUNK)], sbufs[slot], sem.at[slot]).wait()
        pltpu.make_async_copy(ei_hbm.at[1, pl.ds(base, CHUNK)], dbufs[slot], sem.at[slot]).wait()

    def compute_gidx(slot):
        for k in range(CHUNK // 16):
            v = sbufs[slot][pl.ds(k * 16, 16)]
            gbufs[slot][pl.ds(k * 16, 16)] = v + v + c

    def start_gather(xi, xr):
        pltpu.async_copy(xv_hbm.at[gbufs[xi]], rowss[xr], sem.at[NIB + xr])

    def wait_gather(xi, xr):
        pltpu.make_async_copy(xv_hbm.at[gbufs[xi]], rowss[xr], sem.at[NIB + xr]).wait()

    def wait_scatter(jj, xi, xr):
        pltpu.make_async_copy(rowss[xr], acc_sh.at[dbufs[xi]], sem.at[NIB + NRB + xr]).wait()

        @pl.when(cpred(jj))
        def _():
            pltpu.make_async_copy(ones_v, cnt_sh.at[dbufs[xi]], sem.at[NIB + 2 * NRB + xr]).wait()

    def half(j, xi, xr):
        wait_gather(xi, xr)
        pltpu.async_copy(rowss[xr], acc_sh.at[dbufs[xi]], sem.at[NIB + NRB + xr], add=True)

        @pl.when(cpred(j))
        def _():
            pltpu.async_copy(ones_v, cnt_sh.at[dbufs[xi]], sem.at[NIB + 2 * NRB + xr], add=True)

        @pl.when(j >= 2)
        def _():
            wait_scatter(j - 2, (xi + NIB - 2) % NIB, (xr + NRB - 2) % NRB)

        @pl.when(j + 4 < NCHF)
        def _():
            wait_idx(j + 4, (xi + 4) % NIB)
            compute_gidx((xi + 4) % NIB)
            start_gather((xi + 4) % NIB, (xr + 4) % NRB)

        @pl.when(j + 6 < NCHF)
        def _():
            load_idx(j + 6, (xi + 6) % NIB)

    for jj in range(6):
        load_idx(jj, jj)
    for jj in range(4):
        wait_idx(jj, jj)
        compute_gidx(jj)
        start_gather(jj, jj)

    def body(jq, _):
        j = jq * UNROLL
        for u in range(UNROLL):
            half(j + u, u % NIB, u % NRB)
        return 0
    lax.fori_loop(0, NCHF // UNROLL, body, 0)

    for jj in range((NCHF // UNROLL) * UNROLL, NCHF):
        half(jj, jj % NIB, jj % NRB)
    for jj in range(NCHF - 2, NCHF):
        wait_scatter(jj, jj % NIB, jj % NRB)

    rbase = pl.multiple_of(ebase + NCHF * CHUNK, 8)
    pltpu.sync_copy(ei_hbm.at[0, pl.ds(rbase, REM)], srem)
    pltpu.sync_copy(ei_hbm.at[1, pl.ds(rbase, REM)], drem)
    for k in range(REM // 16):
        v = srem[pl.ds(k * 16, 16)]
        grem[pl.ds(k * 16, 16)] = v + v + c
    pltpu.sync_copy(xv_hbm.at[grem], rrem)
    pltpu.sync_copy(rrem, acc_sh.at[drem], add=True)

    @pl.when(c == 0)
    def _():
        pltpu.sync_copy(ones_v.at[pl.ds(0, REM)], cnt_sh.at[drem], add=True)

    plsc.subcore_barrier()

    pltpu.async_copy(acc_sh.at[pl.ds(row0, ROWS_PER_TILE)],
                     acc_out.at[c, pl.ds(row0, ROWS_PER_TILE)], sem.at[NIB + 3 * NRB])
    pltpu.async_copy(cnt_sh.at[pl.ds(row0, ROWS_PER_TILE)],
                     cnt_out.at[c, pl.ds(row0, ROWS_PER_TILE)], sem.at[NIB + 3 * NRB])
    pltpu.make_async_copy(acc_sh.at[pl.ds(row0, ROWS_PER_TILE)],
                          acc_out.at[c, pl.ds(row0, ROWS_PER_TILE)],
                          sem.at[NIB + 3 * NRB]).wait()
    pltpu.make_async_copy(cnt_sh.at[pl.ds(row0, ROWS_PER_TILE)],
                          cnt_out.at[c, pl.ds(row0, ROWS_PER_TILE)],
                          sem.at[NIB + 3 * NRB]).wait()


def _sc_wrap(xv_hbm, ei_hbm, acc_out, cnt_out,
             s0, s1, s2, s3, s4, s5, s6, s7,
             g0, g1, g2, g3, g4, g5, g6, g7,
             d0, d1, d2, d3, d4, d5, d6, d7,
             r0, r1, r2, r3, r4, r5,
             srem, grem, drem, rrem,
             ones_v, zb, zb2, sem, acc_sh, cnt_sh):
    _sc_aggregate_body(xv_hbm, ei_hbm, acc_out, cnt_out,
                       (s0, s1, s2, s3, s4, s5, s6, s7),
                       (g0, g1, g2, g3, g4, g5, g6, g7),
                       (d0, d1, d2, d3, d4, d5, d6, d7),
                       (r0, r1, r2, r3, r4, r5),
                       srem, grem, drem, rrem,
                       ones_v, zb, zb2, sem, acc_sh, cnt_sh)


_sc_aggregate = functools.partial(
    pl.kernel,
    mesh=plsc.VectorSubcoreMesh(core_axis_name="c", subcore_axis_name="s"),
    compiler_params=pltpu.CompilerParams(use_tc_tiling_on_sc=False),
    out_type=[
        jax.ShapeDtypeStruct((NC, N_PAD, DH), jnp.float32),
        jax.ShapeDtypeStruct((NC, N_PAD, CNT_W), jnp.float32),
    ],
    scratch_types=(
        [pltpu.VMEM((CHUNK,), jnp.int32) for _ in range(24)]
        + [pltpu.VMEM((CHUNK, DH), jnp.float32) for _ in range(6)]
        + [pltpu.VMEM((REM,), jnp.int32) for _ in range(3)]
        + [
            pltpu.VMEM((REM, DH), jnp.float32),
            pltpu.VMEM((CHUNK, CNT_W), jnp.float32),
            pltpu.VMEM((ZROWS, DH), jnp.float32),
            pltpu.VMEM((ROWS_PER_TILE, CNT_W), jnp.float32),
            pltpu.SemaphoreType.DMA((NIB + 3 * NRB + 1,)),
            pltpu.VMEM_SHARED((N_PAD, DH), jnp.float32),
            pltpu.VMEM_SHARED((N_PAD, CNT_W), jnp.float32),
        ]
    ),
)(_sc_wrap)


ROWS_PER_BLK = 1000


def _tc_finish_body(acc_ref, cnt_ref, w_ref, b_ref, out_ref):
    s = jnp.concatenate([acc_ref[0], acc_ref[1]], axis=1)
    cnt = cnt_ref[0] + cnt_ref[1]
    deg = cnt[:, 0:1]
    mean = s / jnp.maximum(deg, 1.0)
    h = lax.dot_general(mean, w_ref[...], (((1,), (1,)), ((), ())),
                        preferred_element_type=jnp.float32)
    out_ref[...] = h + b_ref[...]


def _tc_finish(acc, cnt, w, b):
    return pl.pallas_call(
        _tc_finish_body,
        grid=(N_NODES // ROWS_PER_BLK,),
        in_specs=[
            pl.BlockSpec((NC, ROWS_PER_BLK, DH), lambda i: (0, i, 0)),
            pl.BlockSpec((NC, ROWS_PER_BLK, CNT_W), lambda i: (0, i, 0)),
            pl.BlockSpec((D, D), lambda i: (0, 0)),
            pl.BlockSpec((1, D), lambda i: (0, 0)),
        ],
        out_specs=pl.BlockSpec((ROWS_PER_BLK, D), lambda i: (i, 0)),
        out_shape=jax.ShapeDtypeStruct((N_NODES, D), jnp.float32),
    )(acc, cnt, w, b)


@jax.jit
def kernel(x, edge_index, W, b):
    ei = edge_index.astype(jnp.int32)
    xv = x.reshape(2 * N_NODES, DH)
    acc, cnt = _sc_aggregate(xv, ei)
    return _tc_finish(acc, cnt, W, b.reshape(1, D))
